# Initial kernel scaffold; baseline (speedup 1.0000x reference)
import jax
import jax.numpy as jnp
from jax.experimental import pallas as pl


def kernel(x, table):
    raise NotImplementedError("write your pallas kernel here")


# speedup vs baseline: 4.2460x; 4.2460x over previous
import functools

import jax
import jax.numpy as jnp
from jax import lax
from jax.experimental import pallas as pl
from jax.experimental.pallas import tpu as pltpu
from jax.experimental.pallas import tpu_sc as plsc

EMB_DIM = 32
N_TOTAL = 1024 * 26 * 20
NUM_CORES = 2
NUM_SUBCORES = 16
LANES = 16
NUM_WORKERS = NUM_CORES * NUM_SUBCORES
N_PER_W = N_TOTAL // NUM_WORKERS
CHUNK = 128
N_CHUNKS = N_PER_W // CHUNK

_mesh = plsc.VectorSubcoreMesh(core_axis_name="c", subcore_axis_name="s")


@functools.partial(
    pl.kernel,
    mesh=_mesh,
    compiler_params=pltpu.CompilerParams(
        needs_layout_passes=False, use_tc_tiling_on_sc=False),
    out_type=[
        jax.ShapeDtypeStruct((N_TOTAL, EMB_DIM), jnp.float32),
        jax.ShapeDtypeStruct((N_TOTAL,), jnp.float32),
    ],
    scratch_types=[
        pltpu.VMEM((N_PER_W,), jnp.int32),
        pltpu.VMEM((N_PER_W,), jnp.float32),
        pltpu.VMEM((CHUNK, EMB_DIM), jnp.float32),
        pltpu.SemaphoreType.DMA,
    ],
)
def _emb_lookup(x_hbm, table_hbm, out_hbm, mask_hbm, idx_v, mask_v, rows_v, sem):
    wid = lax.axis_index("s") * NUM_CORES + lax.axis_index("c")
    base = wid * N_PER_W
    pltpu.sync_copy(x_hbm.at[pl.ds(base, N_PER_W)], idx_v)

    def mask_body(i, carry):
        v = idx_v[pl.ds(i * LANES, LANES)]
        mask_v[pl.ds(i * LANES, LANES)] = jnp.where(
            v != 0, jnp.float32(1.0), jnp.float32(0.0))
        return carry

    lax.fori_loop(0, N_PER_W // LANES, mask_body, 0, unroll=4)
    pltpu.sync_copy(mask_v, mask_hbm.at[pl.ds(base, N_PER_W)])

    def chunk_body(c, carry):
        off = c * CHUNK
        idx_chunk = idx_v.at[pl.ds(off, CHUNK)]
        pltpu.async_copy(table_hbm.at[idx_chunk], rows_v, sem).wait()
        for g in range(CHUNK // LANES):
            v = idx_v[pl.ds(off + g * LANES, LANES)]
            pad = v == 0
            n_pad = jnp.sum(jnp.where(pad, 1, 0).astype(jnp.int32))

            @pl.when(n_pad > 0)
            def _zero_rows():
                rows = lax.iota(jnp.int32, LANES) + g * LANES
                zeros = jnp.zeros((LANES,), jnp.float32)
                for col in range(EMB_DIM):
                    cols = jnp.full((LANES,), col, jnp.int32)
                    plsc.store_scatter(rows_v, [rows, cols], zeros, mask=pad)

        pltpu.sync_copy(rows_v, out_hbm.at[pl.ds(base + off, CHUNK)])
        return carry

    lax.fori_loop(0, N_CHUNKS, chunk_body, 0)


def kernel(x, table):
    x_flat = x.reshape(-1).astype(jnp.int32)
    out, mask = _emb_lookup(x_flat, table)
    emb = out.reshape(x.shape + (EMB_DIM,))
    return emb, mask.reshape(x.shape)


# speedup vs baseline: 5.4412x; 1.2815x over previous
import functools

import jax
import jax.numpy as jnp
from jax import lax
from jax.experimental import pallas as pl
from jax.experimental.pallas import tpu as pltpu
from jax.experimental.pallas import tpu_sc as plsc

EMB_DIM = 32
N_TOTAL = 1024 * 26 * 20
NUM_CORES = 2
NUM_SUBCORES = 16
LANES = 16
NUM_WORKERS = NUM_CORES * NUM_SUBCORES
N_PER_W = N_TOTAL // NUM_WORKERS
CHUNK = 128
SUP_CHUNKS = 5
SUPER = CHUNK * SUP_CHUNKS
N_SUPERS = N_PER_W // SUPER
N_GROUPS = SUPER // LANES

_mesh = plsc.VectorSubcoreMesh(core_axis_name="c", subcore_axis_name="s")


@functools.partial(
    pl.kernel,
    mesh=_mesh,
    compiler_params=pltpu.CompilerParams(
        needs_layout_passes=False, use_tc_tiling_on_sc=False),
    out_type=[
        jax.ShapeDtypeStruct((N_TOTAL, EMB_DIM), jnp.float32),
        jax.ShapeDtypeStruct((N_TOTAL,), jnp.float32),
    ],
    scratch_types=[
        pltpu.VMEM((N_PER_W,), jnp.int32),
        pltpu.VMEM((N_PER_W,), jnp.float32),
        pltpu.VMEM((SUPER, EMB_DIM), jnp.float32),
        pltpu.VMEM((SUPER, EMB_DIM), jnp.float32),
        pltpu.SemaphoreType.DMA,
        pltpu.SemaphoreType.DMA,
    ],
)
def _emb_lookup(x_hbm, table_hbm, out_hbm, mask_hbm,
                idx_v, mask_v, buf_a, buf_b, sem_a, sem_b):
    wid = lax.axis_index("s") * NUM_CORES + lax.axis_index("c")
    base = wid * N_PER_W
    pltpu.sync_copy(x_hbm.at[pl.ds(base, N_PER_W)], idx_v)

    def issue(s, buf, sem):
        for j in range(SUP_CHUNKS):
            idx_chunk = idx_v.at[pl.ds(s * SUPER + j * CHUNK, CHUNK)]
            pltpu.async_copy(table_hbm.at[idx_chunk],
                             buf.at[pl.ds(j * CHUNK, CHUNK)], sem)

    def drain(buf, sem):
        pltpu.make_async_copy(out_hbm.at[pl.ds(0, SUPER)], buf, sem).wait()

    def process_write(s, buf):
        off = s * SUPER

        def mask_body(g, min_carry):
            v = idx_v[pl.ds(off + g * LANES, LANES)]
            mask_v[pl.ds(off + g * LANES, LANES)] = jnp.where(
                v != 0, jnp.float32(1.0), jnp.float32(0.0))
            return jnp.minimum(min_carry, v)

        minv = lax.fori_loop(0, N_GROUPS, mask_body,
                             jnp.full((LANES,), 1, jnp.int32), unroll=4)

        @pl.when(jnp.min(minv) == 0)
        def _zero_pad_rows():
            def zero_body(g, carry):
                v = idx_v[pl.ds(off + g * LANES, LANES)]
                pad = v == 0
                rows = lax.iota(jnp.int32, LANES) + g * LANES
                zeros = jnp.zeros((LANES,), jnp.float32)
                for col in range(EMB_DIM):
                    cols = jnp.full((LANES,), col, jnp.int32)
                    plsc.store_scatter(buf, [rows, cols], zeros, mask=pad)
                return carry

            lax.fori_loop(0, N_GROUPS, zero_body, 0)

        pltpu.sync_copy(buf, out_hbm.at[pl.ds(base + off, SUPER)])

    issue(0, buf_a, sem_a)

    def super_pair(i, carry):
        s0 = 2 * i
        issue(s0 + 1, buf_b, sem_b)
        drain(buf_a, sem_a)
        process_write(s0, buf_a)

        @pl.when(i < N_SUPERS // 2 - 1)
        def _prefetch_a():
            issue(s0 + 2, buf_a, sem_a)

        drain(buf_b, sem_b)
        process_write(s0 + 1, buf_b)
        return carry

    lax.fori_loop(0, N_SUPERS // 2, super_pair, 0)
    pltpu.sync_copy(mask_v, mask_hbm.at[pl.ds(base, N_PER_W)])


def kernel(x, table):
    x_flat = x.reshape(-1).astype(jnp.int32)
    out, mask = _emb_lookup(x_flat, table)
    emb = out.reshape(x.shape + (EMB_DIM,))
    return emb, mask.reshape(x.shape)
